# baseline (speedup 1.0000x reference)
import functools

import jax
import jax.numpy as jnp
from jax import lax
from jax.experimental import pallas as pl
from jax.experimental.pallas import tpu as pltpu
from jax.experimental.pallas import tpu_sc as plsc

N = 10000
E = 320000
D_IN = 128
D_HID = 128
D_OUT = 64
L = 20000

NC = 2
NS = 16
NW = NC * NS

NP = 10240
EP = 327680
KE = 80
NBE = EP // (NW * KE)
IBE = 16
NGE = NBE // IBE
NG0 = 16
NG1 = 0
NBT = (NG0 + NG1) * IBE
LP = 20480
KP = 128
NBP = LP // (NW * KP)

_mesh = plsc.VectorSubcoreMesh(core_axis_name="c", subcore_axis_name="s")


def _zero_rows(buf, nrows, d):
    z = jnp.zeros((16,), jnp.float32)

    def row(i, _):
        for k in range(d // 16):
            buf[i, pl.ds(k * 16, 16)] = z
        return 0

    lax.fori_loop(0, nrows, row, 0)


def _deg_body(dst_hbm, parts_hbm, deg_sh, idx_v, ones_v, out_v):
    c = lax.axis_index("c")
    s = lax.axis_index("s")
    wid = c * NS + s

    one = jnp.ones((16,), jnp.float32)
    z = jnp.zeros((16,), jnp.float32)

    def fill_ones(i, _):
        ones_v[pl.ds(i * 16, 16)] = one
        return 0

    lax.fori_loop(0, KE // 16, fill_ones, 0)

    def fill_z(i, _):
        out_v[pl.ds(i * 16, 16)] = z
        return 0

    lax.fori_loop(0, (NP // NS) // 16, fill_z, 0)

    pltpu.sync_copy(out_v, deg_sh.at[pl.ds(s * (NP // NS), NP // NS)])
    plsc.subcore_barrier()

    pltpu.sync_copy(dst_hbm.at[wid], idx_v)

    def body(j, _):
        pltpu.sync_copy(ones_v, deg_sh.at[idx_v.at[j]], add=True)
        return 0

    lax.fori_loop(0, NBE, body, 0)
    plsc.subcore_barrier()

    off = s * (NP // NS)
    pltpu.sync_copy(deg_sh.at[pl.ds(off, NP // NS)], out_v)
    pltpu.sync_copy(out_v, parts_hbm.at[c, pl.ds(off, NP // NS)])


_deg_kernel = pl.kernel(
    _deg_body,
    out_type=jax.ShapeDtypeStruct((NC, NP), jnp.float32),
    mesh=_mesh,
    compiler_params=pltpu.CompilerParams(use_tc_tiling_on_sc=False),
    scratch_types=[
        pltpu.VMEM_SHARED((NP,), jnp.float32),
        pltpu.VMEM((NBE, KE), jnp.int32),
        pltpu.VMEM((KE,), jnp.float32),
        pltpu.VMEM((NP // NS,), jnp.float32),
    ],
)


def _make_agg(d):
    def body(y_hbm, src_hbm, dst_hbm, parts_hbm, agg_sh, sidx, didx,
             rows0, rows1, rows2, rows3,
             g0, g1, g2, g3, s0, s1, s2, s3, isem0, isem1):
        c = lax.axis_index("c")
        s = lax.axis_index("s")
        rpt = NP // NS
        rowsL = (rows0, rows1, rows2, rows3)
        gsems = (g0, g1, g2, g3)
        ssems = (s0, s1, s2, s3)
        base = 0

        _zero_rows(rows0, KE, d)
        for i in range(rpt // KE):
            pltpu.sync_copy(rows0, agg_sh.at[pl.ds(s * rpt + i * KE, KE)])
        plsc.subcore_barrier()

        def step(b, p, jl, ssem_wait, refill, rp, rjl):
            rows = rowsL[b]
            bp = (b + 2) % 4
            pltpu.make_async_copy(
                y_hbm.at[sidx.at[p, jl]], rows, gsems[b]).wait()
            pltpu.async_copy(rows, agg_sh.at[didx.at[p, jl]], ssems[b],
                             add=True)
            if refill:
                if ssem_wait:
                    pltpu.make_async_copy(
                        rowsL[bp], agg_sh.at[didx.at[p, jl]],
                        ssems[bp]).wait()
                pltpu.async_copy(
                    y_hbm.at[sidx.at[rp, rjl]], rowsL[bp], gsems[bp])

        def group_body(g, p, is_first, is_last):
            for k in range(4):
                step(k, p, k, not (is_first and k < 2), True, p, k + 2)
            if not is_last:
                nxt = pl.ds(base + (g + 1) * IBE, IBE)
                pltpu.async_copy(src_hbm.at[s, nxt], sidx.at[1 - p], isem0)
                pltpu.async_copy(dst_hbm.at[s, nxt], didx.at[1 - p], isem1)

            def mid(i, _):
                for k in range(4):
                    jl = 4 + 4 * i + k
                    step(k, p, jl, True, True, p, jl + 2)
                return 0

            lax.fori_loop(0, (IBE - 8) // 4, mid, 0)
            if not is_last:
                nxt = pl.ds(base + (g + 1) * IBE, IBE)
                pltpu.make_async_copy(
                    src_hbm.at[s, nxt], sidx.at[1 - p], isem0).wait()
                pltpu.make_async_copy(
                    dst_hbm.at[s, nxt], didx.at[1 - p], isem1).wait()
            for k in range(2):
                step(k, p, IBE - 4 + k, True, True, p, IBE - 2 + k)
            for k in range(2, 4):
                if is_last:
                    step(k, p, IBE - 4 + k, False, False, 0, 0)
                else:
                    step(k, p, IBE - 4 + k, True, True, 1 - p, k - 2)

        @pl.when(c == 0)
        def _edge_pipeline():
            pltpu.sync_copy(src_hbm.at[s, pl.ds(base, IBE)], sidx.at[0])
            pltpu.sync_copy(dst_hbm.at[s, pl.ds(base, IBE)], didx.at[0])
            pltpu.async_copy(y_hbm.at[sidx.at[0, 0]], rows0, g0)
            pltpu.async_copy(y_hbm.at[sidx.at[0, 1]], rows1, g1)

            group_body(0, 0, True, False)

            def pair(t, _):
                gg = 1 + 2 * t
                group_body(gg, 1, False, False)
                group_body(gg + 1, 0, False, False)
                return 0

            lax.fori_loop(0, (NG0 - 2) // 2, pair, 0)
            group_body(NG0 - 1, 1, False, True)

            for b in range(4):
                pltpu.make_async_copy(
                    rowsL[b], agg_sh.at[didx.at[1, IBE - 4 + b]],
                    ssems[b]).wait()

        plsc.subcore_barrier()

        pltpu.sync_copy(agg_sh.at[pl.ds(s * rpt, rpt)],
                        parts_hbm.at[c, pl.ds(s * rpt, rpt)])

    return pl.kernel(
        body,
        out_type=jax.ShapeDtypeStruct((NC, NP, d), jnp.float32),
        mesh=_mesh,
        compiler_params=pltpu.CompilerParams(use_tc_tiling_on_sc=False),
        scratch_types=[
            pltpu.VMEM_SHARED((NP, d), jnp.float32),
            pltpu.VMEM((2, IBE, KE), jnp.int32),
            pltpu.VMEM((2, IBE, KE), jnp.int32),
            pltpu.VMEM((KE, d), jnp.float32),
            pltpu.VMEM((KE, d), jnp.float32),
            pltpu.VMEM((KE, d), jnp.float32),
            pltpu.VMEM((KE, d), jnp.float32),
        ] + [pltpu.SemaphoreType.DMA] * 10,
    )


_agg128 = _make_agg(D_HID)
_agg64 = _make_agg(D_OUT)


def _score_body(x2_hbm, ia_hbm, ib_hbm, pred_hbm, ia, ib, ra, rb, outv, sem):
    c = lax.axis_index("c")
    s = lax.axis_index("s")
    wid = c * NS + s

    pltpu.sync_copy(ia_hbm.at[wid], ia)
    pltpu.sync_copy(ib_hbm.at[wid], ib)

    def batch(j, _):
        pltpu.async_copy(x2_hbm.at[ia.at[j]], ra, sem)
        pltpu.async_copy(x2_hbm.at[ib.at[j]], rb, sem)
        pltpu.make_async_copy(x2_hbm.at[ia.at[j]], ra, sem).wait()
        pltpu.make_async_copy(x2_hbm.at[ib.at[j]], rb, sem).wait()

        for g in range(KP // 16):
            ridx = lax.iota(jnp.int32, 16) + g * 16

            def col(cc, acc):
                for u in range(4):
                    cid = jnp.full((16,), cc * 4 + u, jnp.int32)
                    va = plsc.load_gather(ra, [ridx, cid])
                    vb = plsc.load_gather(rb, [ridx, cid])
                    acc = acc + va * vb
                return acc

            acc = lax.fori_loop(0, D_OUT // 4, col,
                                jnp.zeros((16,), jnp.float32))
            outv[pl.ds(j * KP + g * 16, 16)] = acc
        return 0

    lax.fori_loop(0, NBP, batch, 0)
    pltpu.sync_copy(outv, pred_hbm.at[pl.ds(wid * (LP // NW), LP // NW)])


_score_kernel = pl.kernel(
    _score_body,
    out_type=jax.ShapeDtypeStruct((LP,), jnp.float32),
    mesh=_mesh,
    compiler_params=pltpu.CompilerParams(use_tc_tiling_on_sc=False,
                                         needs_layout_passes=False),
    scratch_types=[
        pltpu.VMEM((NBP, KP), jnp.int32),
        pltpu.VMEM((NBP, KP), jnp.int32),
        pltpu.VMEM((KP, D_OUT), jnp.float32),
        pltpu.VMEM((KP, D_OUT), jnp.float32),
        pltpu.VMEM((LP // NW,), jnp.float32),
        pltpu.SemaphoreType.DMA,
    ],
)


def _mm1_body(degp_ref, x_ref, w1_ref, h_ref, y_ref, dis_ref, dinv_ref):
    deg = degp_ref[0] + degp_ref[1] + 1.0
    dis = lax.rsqrt(deg)
    dinv = 1.0 / deg
    h = jnp.dot(x_ref[...], w1_ref[...], preferred_element_type=jnp.float32)
    h_ref[...] = h
    y_ref[...] = dis * h
    dis_ref[...] = dis
    dinv_ref[...] = dinv


_mm1 = pl.pallas_call(
    _mm1_body,
    out_shape=(
        jax.ShapeDtypeStruct((NP, D_HID), jnp.float32),
        jax.ShapeDtypeStruct((NP, D_HID), jnp.float32),
        jax.ShapeDtypeStruct((NP, 1), jnp.float32),
        jax.ShapeDtypeStruct((NP, 1), jnp.float32),
    ),
)


def _mm2_body(p_ref, h1_ref, dis_ref, dinv_ref, b1_ref, w2_ref,
              h2_ref, y2_ref):
    agg = p_ref[0] + p_ref[1]
    out1 = jax.nn.relu(dis_ref[...] * agg + dinv_ref[...] * h1_ref[...]
                       + b1_ref[...])
    h2 = jnp.dot(out1, w2_ref[...], preferred_element_type=jnp.float32)
    h2_ref[...] = h2
    y2_ref[...] = dis_ref[...] * h2


_mm2 = pl.pallas_call(
    _mm2_body,
    out_shape=(
        jax.ShapeDtypeStruct((NP, D_OUT), jnp.float32),
        jax.ShapeDtypeStruct((NP, D_OUT), jnp.float32),
    ),
)


def _fin_body(p_ref, h2_ref, dis_ref, dinv_ref, b2_ref, x2_ref):
    agg = p_ref[0] + p_ref[1]
    x2_ref[...] = (dis_ref[...] * agg + dinv_ref[...] * h2_ref[...]
                   + b2_ref[...])


_fin = pl.pallas_call(
    _fin_body,
    out_shape=jax.ShapeDtypeStruct((NP, D_OUT), jnp.float32),
)


@jax.jit
def kernel(node_feature, edge_index, edge_label_index, W1, b1, W2, b2):
    ei = edge_index.astype(jnp.int32)
    eli = edge_label_index.astype(jnp.int32)

    x_pad = jnp.zeros((NP, D_IN), jnp.float32).at[:N].set(node_feature)
    srcf = jnp.full((EP,), N, jnp.int32).at[:E].set(ei[0])
    dstf = jnp.full((EP,), N, jnp.int32).at[:E].set(ei[1])
    dst_u = dstf.reshape(NW, NBE, KE)
    src_a = srcf.reshape(NS, NBT, KE)
    dst_a = dstf.reshape(NS, NBT, KE)
    ia = jnp.zeros((LP,), jnp.int32).at[:L].set(eli[0]).reshape(NW, NBP, KP)
    ib = jnp.zeros((LP,), jnp.int32).at[:L].set(eli[1]).reshape(NW, NBP, KP)

    deg_parts = _deg_kernel(dst_u)
    degp = deg_parts.reshape(NC, NP, 1)
    h1, y1, dis, dinv = _mm1(degp, x_pad, W1)
    parts1 = _agg128(y1, src_a, dst_a)
    h2, y2 = _mm2(parts1, h1, dis, dinv, b1.reshape(1, D_HID), W2)
    parts2 = _agg64(y2, src_a, dst_a)
    x2 = _fin(parts2, h2, dis, dinv, b2.reshape(1, D_OUT))
    pred = _score_kernel(x2, ia, ib)
    return pred[:L]


# speedup vs baseline: 1.1906x; 1.1906x over previous
import functools

import jax
import jax.numpy as jnp
from jax import lax
from jax.experimental import pallas as pl
from jax.experimental.pallas import tpu as pltpu
from jax.experimental.pallas import tpu_sc as plsc

N = 10000
E = 320000
D_IN = 128
D_HID = 128
D_OUT = 64
L = 20000

NC = 2
NS = 16
NW = NC * NS

NP = 10240
EP = 327680
KE = 80
NBE = EP // (NW * KE)
IBE = 16
NGE = NBE // IBE
NG0 = 12
NG1 = 4
NBT = (NG0 + NG1) * IBE
LP = 20480
KP = 128
NBP = LP // (NW * KP)

_mesh = plsc.VectorSubcoreMesh(core_axis_name="c", subcore_axis_name="s")


def _zero_rows(buf, nrows, d):
    z = jnp.zeros((16,), jnp.float32)

    def row(i, _):
        for k in range(d // 16):
            buf[i, pl.ds(k * 16, 16)] = z
        return 0

    lax.fori_loop(0, nrows, row, 0)


def _deg_body(dst_hbm, parts_hbm, deg_sh, idx_v, ones_v, out_v):
    c = lax.axis_index("c")
    s = lax.axis_index("s")
    wid = c * NS + s

    one = jnp.ones((16,), jnp.float32)
    z = jnp.zeros((16,), jnp.float32)

    def fill_ones(i, _):
        ones_v[pl.ds(i * 16, 16)] = one
        return 0

    lax.fori_loop(0, KE // 16, fill_ones, 0)

    def fill_z(i, _):
        out_v[pl.ds(i * 16, 16)] = z
        return 0

    lax.fori_loop(0, (NP // NS) // 16, fill_z, 0)

    pltpu.sync_copy(out_v, deg_sh.at[pl.ds(s * (NP // NS), NP // NS)])
    plsc.subcore_barrier()

    pltpu.sync_copy(dst_hbm.at[wid], idx_v)

    def body(j, _):
        pltpu.sync_copy(ones_v, deg_sh.at[idx_v.at[j]], add=True)
        return 0

    lax.fori_loop(0, NBE, body, 0)
    plsc.subcore_barrier()

    off = s * (NP // NS)
    pltpu.sync_copy(deg_sh.at[pl.ds(off, NP // NS)], out_v)
    pltpu.sync_copy(out_v, parts_hbm.at[c, pl.ds(off, NP // NS)])


_deg_kernel = pl.kernel(
    _deg_body,
    out_type=jax.ShapeDtypeStruct((NC, NP), jnp.float32),
    mesh=_mesh,
    compiler_params=pltpu.CompilerParams(use_tc_tiling_on_sc=False),
    scratch_types=[
        pltpu.VMEM_SHARED((NP,), jnp.float32),
        pltpu.VMEM((NBE, KE), jnp.int32),
        pltpu.VMEM((KE,), jnp.float32),
        pltpu.VMEM((NP // NS,), jnp.float32),
    ],
)


def _make_agg(d):
    def body(y_hbm, src_hbm, dst_hbm, parts_hbm, agg_sh, sidx, didx,
             rows0, rows1, rows2, rows3,
             g0, g1, g2, g3, s0, s1, s2, s3, isem0, isem1):
        c = lax.axis_index("c")
        s = lax.axis_index("s")
        rpt = NP // NS
        rowsL = (rows0, rows1, rows2, rows3)
        gsems = (g0, g1, g2, g3)
        ssems = (s0, s1, s2, s3)
        base = c * (NG0 * IBE)
        ng = NG0 + c * (NG1 - NG0)

        _zero_rows(rows0, KE, d)
        for i in range(rpt // KE):
            pltpu.sync_copy(rows0, agg_sh.at[pl.ds(s * rpt + i * KE, KE)])
        plsc.subcore_barrier()

        def step(b, p, jl, ssem_wait, refill, rp, rjl):
            rows = rowsL[b]
            bp = (b + 2) % 4
            pltpu.make_async_copy(
                y_hbm.at[sidx.at[p, jl]], rows, gsems[b]).wait()
            pltpu.async_copy(rows, agg_sh.at[didx.at[p, jl]], ssems[b],
                             add=True)
            if refill:
                if ssem_wait:
                    pltpu.make_async_copy(
                        rowsL[bp], agg_sh.at[didx.at[p, jl]],
                        ssems[bp]).wait()
                pltpu.async_copy(
                    y_hbm.at[sidx.at[rp, rjl]], rowsL[bp], gsems[bp])

        def group_body(g, p, is_first, is_last):
            for k in range(4):
                step(k, p, k, not (is_first and k < 2), True, p, k + 2)
            if not is_last:
                nxt = pl.ds(base + (g + 1) * IBE, IBE)
                pltpu.async_copy(src_hbm.at[s, nxt], sidx.at[1 - p], isem0)
                pltpu.async_copy(dst_hbm.at[s, nxt], didx.at[1 - p], isem1)

            def mid(i, _):
                for k in range(4):
                    jl = 4 + 4 * i + k
                    step(k, p, jl, True, True, p, jl + 2)
                return 0

            lax.fori_loop(0, (IBE - 8) // 4, mid, 0)
            if not is_last:
                nxt = pl.ds(base + (g + 1) * IBE, IBE)
                pltpu.make_async_copy(
                    src_hbm.at[s, nxt], sidx.at[1 - p], isem0).wait()
                pltpu.make_async_copy(
                    dst_hbm.at[s, nxt], didx.at[1 - p], isem1).wait()
            for k in range(2):
                step(k, p, IBE - 4 + k, True, True, p, IBE - 2 + k)
            for k in range(2, 4):
                if is_last:
                    step(k, p, IBE - 4 + k, False, False, 0, 0)
                else:
                    step(k, p, IBE - 4 + k, True, True, 1 - p, k - 2)

        pltpu.sync_copy(src_hbm.at[s, pl.ds(base, IBE)], sidx.at[0])
        pltpu.sync_copy(dst_hbm.at[s, pl.ds(base, IBE)], didx.at[0])
        pltpu.async_copy(y_hbm.at[sidx.at[0, 0]], rows0, g0)
        pltpu.async_copy(y_hbm.at[sidx.at[0, 1]], rows1, g1)

        group_body(0, 0, True, False)

        def pair(t, _):
            gg = 1 + 2 * t
            group_body(gg, 1, False, False)
            group_body(gg + 1, 0, False, False)
            return 0

        lax.fori_loop(0, (ng - 2) // 2, pair, 0)
        group_body(ng - 1, 1, False, True)

        for b in range(4):
            pltpu.make_async_copy(
                rowsL[b], agg_sh.at[didx.at[1, IBE - 4 + b]], ssems[b]).wait()
        plsc.subcore_barrier()

        pltpu.sync_copy(agg_sh.at[pl.ds(s * rpt, rpt)],
                        parts_hbm.at[c, pl.ds(s * rpt, rpt)])

    return pl.kernel(
        body,
        out_type=jax.ShapeDtypeStruct((NC, NP, d), jnp.float32),
        mesh=_mesh,
        compiler_params=pltpu.CompilerParams(use_tc_tiling_on_sc=False),
        scratch_types=[
            pltpu.VMEM_SHARED((NP, d), jnp.float32),
            pltpu.VMEM((2, IBE, KE), jnp.int32),
            pltpu.VMEM((2, IBE, KE), jnp.int32),
            pltpu.VMEM((KE, d), jnp.float32),
            pltpu.VMEM((KE, d), jnp.float32),
            pltpu.VMEM((KE, d), jnp.float32),
            pltpu.VMEM((KE, d), jnp.float32),
        ] + [pltpu.SemaphoreType.DMA] * 10,
    )


_agg128 = _make_agg(D_HID)
_agg64 = _make_agg(D_OUT)


def _score_body(x2_hbm, ia_hbm, ib_hbm, pred_hbm, ia, ib, ra, rb, outv, sem):
    c = lax.axis_index("c")
    s = lax.axis_index("s")
    wid = c * NS + s

    pltpu.sync_copy(ia_hbm.at[wid], ia)
    pltpu.sync_copy(ib_hbm.at[wid], ib)

    def batch(j, _):
        pltpu.async_copy(x2_hbm.at[ia.at[j]], ra, sem)
        pltpu.async_copy(x2_hbm.at[ib.at[j]], rb, sem)
        pltpu.make_async_copy(x2_hbm.at[ia.at[j]], ra, sem).wait()
        pltpu.make_async_copy(x2_hbm.at[ib.at[j]], rb, sem).wait()

        for g in range(KP // 16):
            ridx = lax.iota(jnp.int32, 16) + g * 16

            def col(cc, acc):
                for u in range(4):
                    cid = jnp.full((16,), cc * 4 + u, jnp.int32)
                    va = plsc.load_gather(ra, [ridx, cid])
                    vb = plsc.load_gather(rb, [ridx, cid])
                    acc = acc + va * vb
                return acc

            acc = lax.fori_loop(0, D_OUT // 4, col,
                                jnp.zeros((16,), jnp.float32))
            outv[pl.ds(j * KP + g * 16, 16)] = acc
        return 0

    lax.fori_loop(0, NBP, batch, 0)
    pltpu.sync_copy(outv, pred_hbm.at[pl.ds(wid * (LP // NW), LP // NW)])


_score_kernel = pl.kernel(
    _score_body,
    out_type=jax.ShapeDtypeStruct((LP,), jnp.float32),
    mesh=_mesh,
    compiler_params=pltpu.CompilerParams(use_tc_tiling_on_sc=False,
                                         needs_layout_passes=False),
    scratch_types=[
        pltpu.VMEM((NBP, KP), jnp.int32),
        pltpu.VMEM((NBP, KP), jnp.int32),
        pltpu.VMEM((KP, D_OUT), jnp.float32),
        pltpu.VMEM((KP, D_OUT), jnp.float32),
        pltpu.VMEM((LP // NW,), jnp.float32),
        pltpu.SemaphoreType.DMA,
    ],
)


def _mm1_body(degp_ref, x_ref, w1_ref, h_ref, y_ref, dis_ref, dinv_ref):
    deg = degp_ref[0] + degp_ref[1] + 1.0
    dis = lax.rsqrt(deg)
    dinv = 1.0 / deg
    h = jnp.dot(x_ref[...], w1_ref[...], preferred_element_type=jnp.float32)
    h_ref[...] = h
    y_ref[...] = dis * h
    dis_ref[...] = dis
    dinv_ref[...] = dinv


_mm1 = pl.pallas_call(
    _mm1_body,
    out_shape=(
        jax.ShapeDtypeStruct((NP, D_HID), jnp.float32),
        jax.ShapeDtypeStruct((NP, D_HID), jnp.float32),
        jax.ShapeDtypeStruct((NP, 1), jnp.float32),
        jax.ShapeDtypeStruct((NP, 1), jnp.float32),
    ),
)


def _mm2_body(p_ref, h1_ref, dis_ref, dinv_ref, b1_ref, w2_ref,
              h2_ref, y2_ref):
    agg = p_ref[0] + p_ref[1]
    out1 = jax.nn.relu(dis_ref[...] * agg + dinv_ref[...] * h1_ref[...]
                       + b1_ref[...])
    h2 = jnp.dot(out1, w2_ref[...], preferred_element_type=jnp.float32)
    h2_ref[...] = h2
    y2_ref[...] = dis_ref[...] * h2


_mm2 = pl.pallas_call(
    _mm2_body,
    out_shape=(
        jax.ShapeDtypeStruct((NP, D_OUT), jnp.float32),
        jax.ShapeDtypeStruct((NP, D_OUT), jnp.float32),
    ),
)


def _fin_body(p_ref, h2_ref, dis_ref, dinv_ref, b2_ref, x2_ref):
    agg = p_ref[0] + p_ref[1]
    x2_ref[...] = (dis_ref[...] * agg + dinv_ref[...] * h2_ref[...]
                   + b2_ref[...])


_fin = pl.pallas_call(
    _fin_body,
    out_shape=jax.ShapeDtypeStruct((NP, D_OUT), jnp.float32),
)


@jax.jit
def kernel(node_feature, edge_index, edge_label_index, W1, b1, W2, b2):
    ei = edge_index.astype(jnp.int32)
    eli = edge_label_index.astype(jnp.int32)

    x_pad = jnp.zeros((NP, D_IN), jnp.float32).at[:N].set(node_feature)
    srcf = jnp.full((EP,), N, jnp.int32).at[:E].set(ei[0])
    dstf = jnp.full((EP,), N, jnp.int32).at[:E].set(ei[1])
    dst_u = dstf.reshape(NW, NBE, KE)
    src_a = srcf.reshape(NS, NBT, KE)
    dst_a = dstf.reshape(NS, NBT, KE)
    ia = jnp.zeros((LP,), jnp.int32).at[:L].set(eli[0]).reshape(NW, NBP, KP)
    ib = jnp.zeros((LP,), jnp.int32).at[:L].set(eli[1]).reshape(NW, NBP, KP)

    deg_parts = _deg_kernel(dst_u)
    degp = deg_parts.reshape(NC, NP, 1)
    h1, y1, dis, dinv = _mm1(degp, x_pad, W1)
    parts1 = _agg128(y1, src_a, dst_a)
    h2, y2 = _mm2(parts1, h1, dis, dinv, b1.reshape(1, D_HID), W2)
    parts2 = _agg64(y2, src_a, dst_a)
    x2 = _fin(parts2, h2, dis, dinv, b2.reshape(1, D_OUT))
    pred = _score_kernel(x2, ia, ib)
    return pred[:L]
